# Initial kernel scaffold; baseline (speedup 1.0000x reference)
import jax
import jax.numpy as jnp
from jax import lax
from jax.experimental import pallas as pl
from jax.experimental.pallas import tpu as pltpu
from jax.experimental.pallas import tpu_sc as plsc

VOCAB = 100000
SEQ_LEN = 50
EMBED = 64
BATCH = 4096

_NC = 2
_NS = 16
_NW = _NC * _NS

_B_TOTAL = BATCH * SEQ_LEN
_B_PER_W = _B_TOTAL // _NW
_CHUNK = 128
_N_CHUNKS = _B_PER_W // _CHUNK


def _emb_kernel(x_hbm, table_hbm, out_hbm, idx_v, rows_v, sem):
    wid = lax.axis_index("s") * _NC + lax.axis_index("c")
    base = wid * _B_PER_W
    pltpu.sync_copy(x_hbm.at[wid], idx_v)

    def body(i, carry):
        pltpu.async_copy(table_hbm.at[idx_v.at[i]], rows_v, sem).wait()
        pltpu.sync_copy(rows_v, out_hbm.at[pl.ds(base + i * _CHUNK, _CHUNK)])
        return carry

    lax.fori_loop(0, _N_CHUNKS, body, 0)


@jax.jit
def kernel(x, table):
    x_flat = x.reshape(_NW, _N_CHUNKS, _CHUNK).astype(jnp.int32)
    mesh = plsc.VectorSubcoreMesh(core_axis_name="c", subcore_axis_name="s")
    out = pl.kernel(
        _emb_kernel,
        mesh=mesh,
        out_type=jax.ShapeDtypeStruct((_B_TOTAL, EMBED), jnp.float32),
        scratch_types=[
            pltpu.VMEM((_N_CHUNKS, _CHUNK), jnp.int32),
            pltpu.VMEM((_CHUNK, EMBED), jnp.float32),
            pltpu.SemaphoreType.DMA,
        ],
    )(x_flat, table)
    return out.reshape(BATCH, SEQ_LEN, EMBED)


# speedup vs baseline: 4.0948x; 4.0948x over previous
import jax
import jax.numpy as jnp
from jax import lax
from jax.experimental import pallas as pl
from jax.experimental.pallas import tpu as pltpu
from jax.experimental.pallas import tpu_sc as plsc

VOCAB = 100000
SEQ_LEN = 50
EMBED = 64
BATCH = 4096

_NC = 2
_NS = 16
_NW = _NC * _NS

_B_TOTAL = BATCH * SEQ_LEN
_B_PER_W = _B_TOTAL // _NW
_CHUNK = 128
_N_CHUNKS = _B_PER_W // _CHUNK


def _emb_kernel(x_hbm, table_hbm, out_hbm, idx_v, rows_v, sem):
    wid = lax.axis_index("s") * _NC + lax.axis_index("c")
    base = wid * _B_PER_W
    pltpu.sync_copy(x_hbm.at[wid], idx_v)

    def body(i, carry):
        pltpu.async_copy(table_hbm.at[idx_v.at[i]], rows_v, sem).wait()
        pltpu.sync_copy(rows_v, out_hbm.at[pl.ds(base + i * _CHUNK, _CHUNK)])
        return carry

    lax.fori_loop(0, _N_CHUNKS, body, 0)


@jax.jit
def kernel(x, table):
    x_flat = x.reshape(_NW, _N_CHUNKS, _CHUNK).astype(jnp.int32)
    mesh = plsc.VectorSubcoreMesh(core_axis_name="c", subcore_axis_name="s")
    out = pl.kernel(
        _emb_kernel,
        mesh=mesh,
        out_type=jax.ShapeDtypeStruct((_B_TOTAL, EMBED), jnp.float32),
        scratch_types=[
            pltpu.VMEM((_N_CHUNKS, _CHUNK), jnp.int32),
            pltpu.VMEM((_CHUNK, EMBED), jnp.float32),
            pltpu.SemaphoreType.DMA,
        ],
        compiler_params=pltpu.CompilerParams(use_tc_tiling_on_sc=False),
    )(x_flat, table)
    return out.reshape(BATCH, SEQ_LEN, EMBED)


# speedup vs baseline: 4.6157x; 1.1272x over previous
import jax
import jax.numpy as jnp
from jax import lax
from jax.experimental import pallas as pl
from jax.experimental.pallas import tpu as pltpu
from jax.experimental.pallas import tpu_sc as plsc

VOCAB = 100000
SEQ_LEN = 50
EMBED = 64
BATCH = 4096

_NC = 2
_NS = 16
_NW = _NC * _NS

_B_TOTAL = BATCH * SEQ_LEN
_B_PER_W = _B_TOTAL // _NW
_CHUNK = 128
_N_CHUNKS = _B_PER_W // _CHUNK
_K = 5
_BIG = _K * _CHUNK
_N_BIG = _N_CHUNKS // _K


def _emb_kernel(x_hbm, table_hbm, out_hbm, idx_v, rows_v, sem0, sem1):
    wid = lax.axis_index("s") * _NC + lax.axis_index("c")
    base = wid * _B_PER_W
    pltpu.sync_copy(x_hbm.at[wid], idx_v)

    sems = (sem0, sem1)

    def fire(j, b):
        for t in range(_K):
            pltpu.async_copy(
                table_hbm.at[idx_v.at[j * _K + t]],
                rows_v.at[b, pl.ds(t * _CHUNK, _CHUNK)],
                sems[b],
            )

    def drain(j, b):
        for t in range(_K):
            pltpu.make_async_copy(
                table_hbm.at[idx_v.at[j * _K + t]],
                rows_v.at[b, pl.ds(t * _CHUNK, _CHUNK)],
                sems[b],
            ).wait()

    fire(0, 0)

    def outer(g, carry):
        for b in range(2):
            i = 2 * g + b
            drain(i, b)

            @pl.when(i + 1 < _N_BIG)
            def _():
                fire(i + 1, 1 - b)

            pltpu.sync_copy(
                rows_v.at[b], out_hbm.at[pl.ds(base + i * _BIG, _BIG)]
            )
        return carry

    lax.fori_loop(0, _N_BIG // 2, outer, 0)


@jax.jit
def kernel(x, table):
    x_flat = x.reshape(_NW, _N_CHUNKS, _CHUNK).astype(jnp.int32)
    mesh = plsc.VectorSubcoreMesh(core_axis_name="c", subcore_axis_name="s")
    out = pl.kernel(
        _emb_kernel,
        mesh=mesh,
        out_type=jax.ShapeDtypeStruct((_B_TOTAL, EMBED), jnp.float32),
        scratch_types=[
            pltpu.VMEM((_N_CHUNKS, _CHUNK), jnp.int32),
            pltpu.VMEM((2, _BIG, EMBED), jnp.float32),
            pltpu.SemaphoreType.DMA,
            pltpu.SemaphoreType.DMA,
        ],
        compiler_params=pltpu.CompilerParams(use_tc_tiling_on_sc=False),
    )(x_flat, table)
    return out.reshape(BATCH, SEQ_LEN, EMBED)
